# baseline (speedup 1.0000x reference)
import functools

import jax
import jax.numpy as jnp
from jax import lax
from jax.experimental import pallas as pl
from jax.experimental.pallas import tpu as pltpu
from jax.experimental.pallas import tpu_sc as plsc

N = 10000
D = 256
H = 512
K = 5000
NP = 10240
KP = 5120
BI = 256
BJ = 1024
NEG = float("-inf")


def _rank_body(s_cols_ref, s_rows_ref, ranks_ref):
    pid = pl.program_id(0)
    i0 = pid * BI
    iids = i0 + lax.broadcasted_iota(jnp.int32, (BI, 1), 0)
    pid_div = pid // (BJ // BI)
    cols = []
    for c in range(3):
        si = s_cols_ref[:, c:c + 1]
        acc = jnp.zeros((BI, 1), jnp.float32)
        for jc in range(NP // BJ):
            sj = s_rows_ref[c:c + 1, jc * BJ:(jc + 1) * BJ]

            def _ge(sj=sj, si=si):
                return jnp.sum(jnp.where(sj >= si, 1.0, 0.0), axis=1,
                               keepdims=True)

            def _gt(sj=sj, si=si):
                return jnp.sum(jnp.where(sj > si, 1.0, 0.0), axis=1,
                               keepdims=True)

            def _diag(sj=sj, si=si, jc=jc):
                jids = jc * BJ + lax.broadcasted_iota(jnp.int32, (1, BJ), 1)
                ge_f = jnp.where(sj >= si, 1.0, 0.0)
                gt_f = jnp.where(sj > si, 1.0, 0.0)
                contrib = jnp.where(jids < iids, ge_f, gt_f)
                return jnp.sum(contrib, axis=1, keepdims=True)

            part = lax.cond(
                jc < pid_div, _ge,
                lambda _diag=_diag, _gt=_gt, jc=jc: lax.cond(
                    jc == pid_div, _diag, _gt))
            acc = acc + part
        cols.append(acc)
    rank = jnp.concatenate(
        cols + [jnp.zeros((BI, 1), jnp.float32)] * 5, axis=1).astype(jnp.int32)
    ranks_ref[...] = jnp.where(iids < N, rank, iids)


def _compute_ranks(s_cols, s_rows):
    return pl.pallas_call(
        _rank_body,
        grid=(NP // BI,),
        in_specs=[
            pl.BlockSpec((BI, 8), lambda i: (i, 0)),
            pl.BlockSpec((8, NP), lambda i: (0, 0)),
        ],
        out_specs=pl.BlockSpec((BI, 8), lambda i: (i, 0)),
        out_shape=jax.ShapeDtypeStruct((NP, 8), jnp.int32),
    )(s_cols, s_rows)


def _sc_mesh():
    return plsc.VectorSubcoreMesh(core_axis_name="c", subcore_axis_name="s")


_NW = 32
_SCHUNK = NP // _NW
_SSEG = 80
_SNSEG = _SCHUNK // _SSEG
_GROWS = KP // _NW
_GSEG = 80
_GNSEG = _GROWS // _GSEG


def _scatter_build(r0, r1, r2, s0, s1, s2, rowids):
    @functools.partial(
        pl.kernel,
        mesh=_sc_mesh(),
        out_type=[jax.ShapeDtypeStruct((NP,), jnp.int32)] * 3
                 + [jax.ShapeDtypeStruct((NP,), jnp.float32)] * 3,
        scratch_types=[
            pltpu.VMEM((_SNSEG, _SSEG), jnp.int32),
            pltpu.VMEM((_SNSEG, _SSEG), jnp.int32),
            pltpu.VMEM((_SNSEG, _SSEG), jnp.float32),
            pltpu.SemaphoreType.DMA,
            pltpu.SemaphoreType.DMA,
        ],
    )
    def k(r0_h, r1_h, r2_h, s0_h, s1_h, s2_h, ids_h,
          p0_h, p1_h, p2_h, v0_h, v1_h, v2_h,
          rk_v, ids_v, sv_v, sem_a, sem_b):
        wid = lax.axis_index("s") * 2 + lax.axis_index("c")
        base = wid * _SCHUNK
        for g in range(_SNSEG):
            pltpu.sync_copy(ids_h.at[pl.ds(base + g * _SSEG, _SSEG)],
                            ids_v.at[g])
        for r_h, s_h, p_h, v_h in ((r0_h, s0_h, p0_h, v0_h),
                                   (r1_h, s1_h, p1_h, v1_h),
                                   (r2_h, s2_h, p2_h, v2_h)):
            for g in range(_SNSEG):
                pltpu.sync_copy(r_h.at[pl.ds(base + g * _SSEG, _SSEG)],
                                rk_v.at[g])
                pltpu.sync_copy(s_h.at[pl.ds(base + g * _SSEG, _SSEG)],
                                sv_v.at[g])
            cps = []
            for g in range(_SNSEG):
                cps.append(pltpu.async_copy(ids_v.at[g], p_h.at[rk_v.at[g]],
                                            sem_a))
                cps.append(pltpu.async_copy(sv_v.at[g], v_h.at[rk_v.at[g]],
                                            sem_b))
            for cp in cps:
                cp.wait()

    return k(r0, r1, r2, s0, s1, s2, rowids)


def _gather_rows(x, p0, p1, p2):
    @functools.partial(
        pl.kernel,
        mesh=_sc_mesh(),
        out_type=[jax.ShapeDtypeStruct((KP, D), jnp.float32)] * 3,
        scratch_types=[
            pltpu.VMEM((_GNSEG, _GSEG), jnp.int32),
            pltpu.VMEM((_GROWS, D), jnp.float32),
            pltpu.SemaphoreType.DMA,
        ],
    )
    def k(x_h, p0_h, p1_h, p2_h, o0_h, o1_h, o2_h, idx_v, rows_v, sem):
        wid = lax.axis_index("s") * 2 + lax.axis_index("c")
        base = wid * _GROWS
        for p_h, o_h in ((p0_h, o0_h), (p1_h, o1_h), (p2_h, o2_h)):
            for g in range(_GNSEG):
                pltpu.sync_copy(p_h.at[pl.ds(base + g * _GSEG, _GSEG)],
                                idx_v.at[g])
            cps = [pltpu.async_copy(x_h.at[idx_v.at[g]],
                                    rows_v.at[pl.ds(g * _GSEG, _GSEG)], sem)
                   for g in range(_GNSEG)]
            for cp in cps:
                cp.wait()
            pltpu.sync_copy(rows_v, o_h.at[pl.ds(base, _GROWS)])

    return k(x, p0, p1, p2)


_BM = 256


def _mlp_body(p0_ref, p1_ref, p2_ref, tv_ref, w1a_ref, w1b_ref, w1c_ref,
              b1_ref, w2_ref, b2_ref, out_ref):
    tv = jnp.tanh(tv_ref[...])
    acc = jnp.dot(p0_ref[...] * tv[:, 0:1], w1a_ref[...],
                  preferred_element_type=jnp.float32)
    acc += jnp.dot(p1_ref[...] * tv[:, 1:2], w1b_ref[...],
                   preferred_element_type=jnp.float32)
    acc += jnp.dot(p2_ref[...] * tv[:, 2:3], w1c_ref[...],
                   preferred_element_type=jnp.float32)
    h = jnp.maximum(acc + b1_ref[0:1, :], 0.0)
    out_ref[...] = jnp.dot(h, w2_ref[...],
                           preferred_element_type=jnp.float32) + b2_ref[0:1, :]


def _mlp(g0, g1, g2, tvc8, W1a, W1b, W1c, b1t, W2, b2t):
    return pl.pallas_call(
        _mlp_body,
        grid=(KP // _BM,),
        in_specs=[
            pl.BlockSpec((_BM, D), lambda i: (i, 0)),
            pl.BlockSpec((_BM, D), lambda i: (i, 0)),
            pl.BlockSpec((_BM, D), lambda i: (i, 0)),
            pl.BlockSpec((_BM, 8), lambda i: (i, 0)),
            pl.BlockSpec((D, H), lambda i: (0, 0)),
            pl.BlockSpec((D, H), lambda i: (0, 0)),
            pl.BlockSpec((D, H), lambda i: (0, 0)),
            pl.BlockSpec((8, H), lambda i: (0, 0)),
            pl.BlockSpec((H, H), lambda i: (0, 0)),
            pl.BlockSpec((8, H), lambda i: (0, 0)),
        ],
        out_specs=pl.BlockSpec((_BM, H), lambda i: (i, 0)),
        out_shape=jax.ShapeDtypeStruct((KP, H), jnp.float32),
    )(g0, g1, g2, tvc8, W1a, W1b, W1c, b1t, W2, b2t)


def kernel(x, edge_index, batch, w0, w1, w2, W1, b1, W2, b2):
    s_list = [x @ w / (jnp.linalg.norm(w) + 1e-16) for w in (w0, w1, w2)]
    pad = jnp.full((NP - N,), NEG, jnp.float32)
    s_pad = [jnp.concatenate([s, pad]) for s in s_list]
    s_cols8 = jnp.pad(jnp.stack(s_pad, axis=1), ((0, 0), (0, 5)))
    s_rows8 = jnp.pad(jnp.stack(s_pad, axis=0), ((0, 5), (0, 0)))

    ranks8 = _compute_ranks(s_cols8, s_rows8)

    rowids = jnp.arange(NP, dtype=jnp.int32)
    p0, p1, p2, v0, v1, v2 = _scatter_build(
        ranks8[:, 0], ranks8[:, 1], ranks8[:, 2],
        s_pad[0], s_pad[1], s_pad[2], rowids)

    g0, g1, g2 = _gather_rows(x, p0, p1, p2)

    tvc8 = jnp.pad(jnp.stack([v0[:KP], v1[:KP], v2[:KP]], axis=1),
                   ((0, 0), (0, 5)))
    b1t = jnp.broadcast_to(b1[None, :], (8, H))
    b2t = jnp.broadcast_to(b2[None, :], (8, H))
    fused_p = _mlp(g0, g1, g2, tvc8, W1[:D], W1[D:2 * D], W1[2 * D:],
                   b1t, W2, b2t)
    return fused_p[:K], edge_index, batch
